# baseline (speedup 1.0000x reference)
import functools

import jax
import jax.numpy as jnp
import numpy as np
from jax.experimental import pallas as pl
from jax.experimental.pallas import tpu as pltpu

S = 30.0
M = 0.5
ARC_START_EPOCH = 1
GAMMA = 2.0
COS_M = float(np.cos(M))
SIN_M = float(np.sin(M))
BORDER = float(np.cos(np.pi - M))


def _loss_body(use_arc_ref, scale_ref, fc_ref, label_ref, focal_ref, acc_ref,
               lacc_ref, cacc_ref, *, inv_n, rows, rsub, unroll):
    i = pl.program_id(0)
    nsteps = pl.num_programs(0)
    use_arc = use_arc_ref[0, 0] != 0
    scale = scale_ref[0, 0]

    @pl.when(i == 0)
    def _init():
        lacc_ref[...] = jnp.zeros_like(lacc_ref)
        cacc_ref[...] = jnp.zeros_like(cacc_ref)

    def tile(k):
        c = fc_ref[pl.ds(k * rsub, rsub), :]
        t = label_ref[pl.ds(k * rsub, rsub), :]

        sin_t = jnp.sqrt(jnp.maximum(1.0 - c * c, 0.0))
        phai = c * COS_M - sin_t * SIN_M
        phai = jnp.where(c > BORDER, phai, -2.0 - phai)

        tmask = t != 0.0
        inner = jnp.where(use_arc, phai, c)
        v = scale * jnp.where(tmask, -inner, c)

        q = jnp.exp(jnp.minimum(v, -v))
        sp = jnp.maximum(v, 0.0) + jnp.log1p(q)
        loss = jnp.exp(2.0 * (v - sp)) * sp

        corr = jnp.where(v < 0.0, 1.0, 0.0)
        return loss, corr

    def group_step(g, carry):
        parts = [tile(g * unroll + j) for j in range(unroll)]
        ls = [p[0] for p in parts]
        cs = [p[1] for p in parts]
        while len(ls) > 1:
            ls = [a + b for a, b in zip(ls[::2], ls[1::2])]
            cs = [a + b for a, b in zip(cs[::2], cs[1::2])]
        lacc_ref[...] += ls[0]
        cacc_ref[...] += cs[0]
        return carry

    jax.lax.fori_loop(0, rows // rsub // unroll, group_step, 0)

    @pl.when(i == nsteps - 1)
    def _fin():
        focal_ref[0, 0] = jnp.sum(lacc_ref[...]) * inv_n
        acc_ref[0, 0] = jnp.sum(cacc_ref[...]) * inv_n


def kernel(fc, label, epoch):
    B, C = fc.shape
    BR = 512
    RSUB = 8
    UNROLL = 8
    nb = B // BR
    use_arc = (jnp.asarray(epoch, jnp.int32) >= ARC_START_EPOCH).astype(jnp.int32)
    scale = jnp.where(use_arc != 0, jnp.float32(S), jnp.float32(1.0))

    focal2d, acc2d = pl.pallas_call(
        functools.partial(_loss_body, inv_n=1.0 / (B * C),
                          rows=BR, rsub=RSUB, unroll=UNROLL),
        grid=(nb,),
        in_specs=[
            pl.BlockSpec(memory_space=pltpu.SMEM),
            pl.BlockSpec(memory_space=pltpu.SMEM),
            pl.BlockSpec((BR, C), lambda i: (0, 0)),
            pl.BlockSpec((BR, C), lambda i: (0, 0)),
        ],
        out_specs=[
            pl.BlockSpec(memory_space=pltpu.SMEM),
            pl.BlockSpec(memory_space=pltpu.SMEM),
        ],
        out_shape=[
            jax.ShapeDtypeStruct((1, 1), jnp.float32),
            jax.ShapeDtypeStruct((1, 1), jnp.float32),
        ],
        scratch_shapes=[
            pltpu.VMEM((RSUB, C), jnp.float32),
            pltpu.VMEM((RSUB, C), jnp.float32),
        ],
    )(use_arc.reshape(1, 1), scale.reshape(1, 1), fc, label)

    focal = focal2d[0, 0]
    acc = acc2d[0, 0]
    return (focal, acc, focal)


# speedup vs baseline: 1.0532x; 1.0532x over previous
import functools

import jax
import jax.numpy as jnp
import numpy as np
from jax.experimental import pallas as pl
from jax.experimental.pallas import tpu as pltpu

S = 30.0
M = 0.5
ARC_START_EPOCH = 1
GAMMA = 2.0
COS_M = float(np.cos(M))
SIN_M = float(np.sin(M))
BORDER = float(np.cos(np.pi - M))


def _loss_body(use_arc_ref, scale_ref, fc_ref, label_ref, focal_ref, acc_ref,
               lacc_ref, cacc_ref, *, inv_n, rows, rsub, unroll):
    i = pl.program_id(0)
    nsteps = pl.num_programs(0)
    use_arc = use_arc_ref[0, 0] != 0
    scale = scale_ref[0, 0]

    @pl.when(i == 0)
    def _init():
        lacc_ref[...] = jnp.zeros_like(lacc_ref)
        cacc_ref[...] = jnp.zeros_like(cacc_ref)

    def tile(k):
        c = fc_ref[pl.ds(k * rsub, rsub), :]
        t = label_ref[pl.ds(k * rsub, rsub), :]

        sin_t = jnp.sqrt(jnp.maximum(1.0 - c * c, 0.0))
        phai = c * COS_M - sin_t * SIN_M
        phai = jnp.where(c > BORDER, phai, -2.0 - phai)

        tmask = t != 0.0
        inner = jnp.where(use_arc, phai, c)
        v = scale * jnp.where(tmask, -inner, c)

        log2e = 1.4426950408889634
        ln2 = 0.6931471805599453
        q = jnp.exp2(jnp.minimum(v, -v) * log2e)
        sp = jnp.maximum(v, 0.0) + jnp.log2(1.0 + q) * ln2
        loss = jnp.exp2((2.0 * log2e) * (v - sp)) * sp

        corr = jnp.where(v < 0.0, 1.0, 0.0)
        return loss, corr

    def group_step(g, carry):
        parts = [tile(g * unroll + j) for j in range(unroll)]
        ls = [p[0] for p in parts]
        cs = [p[1] for p in parts]
        while len(ls) > 1:
            ls = [a + b for a, b in zip(ls[::2], ls[1::2])]
            cs = [a + b for a, b in zip(cs[::2], cs[1::2])]
        lacc_ref[...] += ls[0]
        cacc_ref[...] += cs[0]
        return carry

    jax.lax.fori_loop(0, rows // rsub // unroll, group_step, 0)

    @pl.when(i == nsteps - 1)
    def _fin():
        focal_ref[0, 0] = jnp.sum(lacc_ref[...]) * inv_n
        acc_ref[0, 0] = jnp.sum(cacc_ref[...]) * inv_n


def kernel(fc, label, epoch):
    B, C = fc.shape
    BR = 512
    RSUB = 8
    UNROLL = 16
    nb = B // BR
    use_arc = (jnp.asarray(epoch, jnp.int32) >= ARC_START_EPOCH).astype(jnp.int32)
    scale = jnp.where(use_arc != 0, jnp.float32(S), jnp.float32(1.0))

    focal2d, acc2d = pl.pallas_call(
        functools.partial(_loss_body, inv_n=1.0 / (B * C),
                          rows=BR, rsub=RSUB, unroll=UNROLL),
        grid=(nb,),
        in_specs=[
            pl.BlockSpec(memory_space=pltpu.SMEM),
            pl.BlockSpec(memory_space=pltpu.SMEM),
            pl.BlockSpec((BR, C), lambda i: (i, 0)),
            pl.BlockSpec((BR, C), lambda i: (i, 0)),
        ],
        out_specs=[
            pl.BlockSpec(memory_space=pltpu.SMEM),
            pl.BlockSpec(memory_space=pltpu.SMEM),
        ],
        out_shape=[
            jax.ShapeDtypeStruct((1, 1), jnp.float32),
            jax.ShapeDtypeStruct((1, 1), jnp.float32),
        ],
        scratch_shapes=[
            pltpu.VMEM((RSUB, C), jnp.float32),
            pltpu.VMEM((RSUB, C), jnp.float32),
        ],
    )(use_arc.reshape(1, 1), scale.reshape(1, 1), fc, label)

    focal = focal2d[0, 0]
    acc = acc2d[0, 0]
    return (focal, acc, focal)
